# baseline (speedup 1.0000x reference)
import functools

import jax
import jax.numpy as jnp
from jax import lax
from jax.experimental import pallas as pl
from jax.experimental.pallas import tpu as pltpu
from jax.experimental.pallas import tpu_sc as plsc

N = 10000
E = 320000
D = 128
G = 64
NCLS = 10

N_PAD = 10240
TRASH = N
NC, NS, L = 2, 16, 16
NW = NC * NS
N_ACC = 10240
RPA = N_ACC // NS
NBUF = 2

CH = 128
NCH = 80
SBC = 8
NSB = NCH // SBC
EPW = CH * NCH
E_PAD = EPW * NW

BN = 1024
GRID = N_PAD // BN

_mesh = plsc.VectorSubcoreMesh(
    core_axis_name="c", subcore_axis_name="s", num_cores=NC, num_subcores=NS)


def _zero_stripe(acc_sh, zbuf, zrows, sid):
    full, rem = RPA // zrows, RPA % zrows
    for k in range(full):
        pltpu.sync_copy(zbuf, acc_sh.at[pl.ds(sid * RPA + k * zrows, zrows)])
    if rem:
        pltpu.sync_copy(zbuf.at[pl.ds(0, rem)],
                        acc_sh.at[pl.ds(sid * RPA + full * zrows, rem)])


def _writeback(acc_sh, out_hbm, cid, sid):
    pltpu.sync_copy(acc_sh.at[pl.ds(sid * RPA, RPA)],
                    out_hbm.at[pl.ds(cid * N_PAD + sid * RPA, RPA)])


@functools.partial(
    pl.kernel,
    out_type=jax.ShapeDtypeStruct((NC * N_PAD, D), jnp.float32),
    mesh=_mesh,
    scratch_types=[
        pltpu.VMEM((NCH, CH), jnp.int32),
        pltpu.VMEM((CH, D), jnp.float32),
        pltpu.VMEM_SHARED((N_ACC, D), jnp.float32),
    ] + [pltpu.SemaphoreType.DMA] * NBUF,
)
def _sc_degree(dst_hbm, zeros_hbm, ones_hbm, out_hbm, dst_v, buf_v, acc_sh,
               *ss):
    cid = lax.axis_index("c")
    sid = lax.axis_index("s")
    wid = sid * NC + cid

    pltpu.sync_copy(dst_hbm.at[wid], dst_v)
    pltpu.sync_copy(zeros_hbm, buf_v)
    _zero_stripe(acc_sh, buf_v, CH, sid)
    plsc.subcore_barrier()
    pltpu.sync_copy(ones_hbm, buf_v)

    for b in range(NBUF):
        pltpu.async_copy(buf_v, acc_sh.at[dst_v.at[b]], ss[b], add=True)

    @pl.loop(NBUF, NCH, step=NBUF)
    def _(j0):
        for b in range(NBUF):
            j = j0 + b
            pltpu.make_async_copy(buf_v, acc_sh.at[dst_v.at[j - NBUF]],
                                  ss[b]).wait()
            pltpu.async_copy(buf_v, acc_sh.at[dst_v.at[j]], ss[b], add=True)

    for b in range(NBUF):
        pltpu.make_async_copy(buf_v, acc_sh.at[dst_v.at[NCH - NBUF + b]],
                              ss[b]).wait()
    plsc.subcore_barrier()
    _writeback(acc_sh, out_hbm, cid, sid)


@functools.partial(
    pl.kernel,
    out_type=jax.ShapeDtypeStruct((NC * N_PAD, D), jnp.float32),
    mesh=_mesh,
    scratch_types=[
        pltpu.VMEM((2, SBC, CH), jnp.int32),
        pltpu.VMEM((2, SBC, CH), jnp.int32),
        pltpu.VMEM((CH, D), jnp.float32),
        pltpu.VMEM((CH, D), jnp.float32),
        pltpu.VMEM_SHARED((N_ACC, D), jnp.float32),
    ] + [pltpu.SemaphoreType.DMA] * 5,
)
def _sc_segsum(g_hbm, src_hbm, dst_hbm, zeros_hbm, out_hbm, src_v, dst_v,
               buf0, buf1, acc_sh, sg0, sg1, ss0, ss1, si):
    cid = lax.axis_index("c")
    sid = lax.axis_index("s")
    wid = sid * NC + cid
    bufs = (buf0, buf1)
    sg = (sg0, sg1)
    ss = (ss0, ss1)

    pltpu.sync_copy(src_hbm.at[wid, 0], src_v.at[0])
    pltpu.sync_copy(dst_hbm.at[wid, 0], dst_v.at[0])
    pltpu.async_copy(src_hbm.at[wid, 1], src_v.at[1], si)
    pltpu.async_copy(dst_hbm.at[wid, 1], dst_v.at[1], si)
    pltpu.async_copy(g_hbm.at[src_v.at[0, 0]], buf0, sg0)
    pltpu.sync_copy(zeros_hbm, buf1)
    _zero_stripe(acc_sh, buf1, CH, sid)
    plsc.subcore_barrier()
    pltpu.async_copy(g_hbm.at[src_v.at[0, 1]], buf1, sg1)
    for b in range(NBUF):
        pltpu.make_async_copy(g_hbm.at[src_v.at[0, b]], bufs[b], sg[b]).wait()
        pltpu.async_copy(bufs[b], acc_sh.at[dst_v.at[0, b]], ss[b], add=True)

    @pl.loop(NBUF, NCH, step=NBUF)
    def _(j0):
        sb = j0 // SBC
        jj = j0 % SBC
        p = sb % 2

        @pl.when(jj == 0)
        def _():
            pltpu.make_async_copy(src_hbm.at[wid, 0], src_v.at[0], si).wait()
            pltpu.make_async_copy(dst_hbm.at[wid, 0], dst_v.at[0], si).wait()

        @pl.when((jj == 0) & (sb < NSB - 1))
        def _():
            pltpu.async_copy(src_hbm.at[wid, sb + 1], src_v.at[1 - p], si)
            pltpu.async_copy(dst_hbm.at[wid, sb + 1], dst_v.at[1 - p], si)

        for b in range(NBUF):
            pltpu.make_async_copy(bufs[b], acc_sh.at[dst_v.at[0, 0]],
                                  ss[b]).wait()
            pltpu.async_copy(g_hbm.at[src_v.at[p, jj + b]], bufs[b], sg[b])
        for b in range(NBUF):
            pltpu.make_async_copy(g_hbm.at[src_v.at[p, jj + b]], bufs[b],
                                  sg[b]).wait()
            pltpu.async_copy(bufs[b], acc_sh.at[dst_v.at[p, jj + b]], ss[b],
                             add=True)

    for b in range(NBUF):
        pltpu.make_async_copy(bufs[b], acc_sh.at[dst_v.at[0, 0]], ss[b]).wait()
    plsc.subcore_barrier()
    _writeback(acc_sh, out_hbm, cid, sid)


def _tc_mm(x_ref, w_ref, o_ref):
    o_ref[...] = jnp.dot(x_ref[...], w_ref[...],
                         preferred_element_type=jnp.float32)


def _tc_scale(xw_ref, d0_ref, d1_ref, g_ref, dis_ref):
    deg = d0_ref[:, :1] + d1_ref[:, :1] + 1.0
    dis = lax.rsqrt(deg)
    dis_ref[...] = dis
    g_ref[...] = dis * xw_ref[...]


def _edge_sum(a0_ref, a1_ref, g_ref):
    return a0_ref[...] + a1_ref[...] + g_ref[...]


def _tc_layer(a0_ref, a1_ref, g_ref, dis_ref, b_ref, w_ref, h_ref, gn_ref):
    dis = dis_ref[...]
    h = jnp.maximum(
        dis * _edge_sum(a0_ref, a1_ref, g_ref) + b_ref[...], 0.0)
    h_ref[...] = h
    gn_ref[...] = dis * jnp.dot(h, w_ref[...],
                                preferred_element_type=jnp.float32)


def _tc_layer_nox(a0_ref, a1_ref, g_ref, dis_ref, b_ref, w_ref, gn_ref):
    dis = dis_ref[...]
    h = jnp.maximum(
        dis * _edge_sum(a0_ref, a1_ref, g_ref) + b_ref[...], 0.0)
    gn_ref[...] = dis * jnp.dot(h, w_ref[...],
                                preferred_element_type=jnp.float32)


def _tc_pool(a0_ref, a1_ref, g_ref, dis_ref, b_ref, x1_ref, x2_ref, bt_ref,
             lw_ref, lb_ref, out_ref, sums, cnts):
    i = pl.program_id(0)
    x3 = jnp.maximum(
        dis_ref[...] * _edge_sum(a0_ref, a1_ref, g_ref) + b_ref[...], 0.0)
    xs = jnp.concatenate([x1_ref[...], x2_ref[...], x3], axis=1)
    oh = (lax.broadcasted_iota(jnp.int32, (G, BN), 0)
          == bt_ref[...]).astype(jnp.float32)
    s_c = jnp.dot(oh, xs, preferred_element_type=jnp.float32)
    c_c = jnp.sum(oh, axis=1, keepdims=True)

    @pl.when(i == 0)
    def _():
        sums[...] = s_c
        cnts[...] = c_c

    @pl.when(i > 0)
    def _():
        sums[...] += s_c
        cnts[...] += c_c

    @pl.when(i == GRID - 1)
    def _():
        pooled = sums[...] / jnp.maximum(cnts[...], 1.0)
        logits = jnp.dot(pooled, lw_ref[...],
                         preferred_element_type=jnp.float32) + lb_ref[...]
        m = jnp.max(logits, axis=1, keepdims=True)
        lse = jnp.log(jnp.sum(jnp.exp(logits - m), axis=1, keepdims=True)) + m
        out_ref[...] = logits - lse


def _row_spec(off):
    return pl.BlockSpec((BN, D), lambda i, o=off: (i + o, 0))


def _full_spec(shape):
    return pl.BlockSpec(shape, lambda i: (0, 0))


def kernel(x, edge_index, batch, W1, b1, W2, b2, W3, b3, W4, b4, lin_W, lin_b):
    f32 = jnp.float32
    x_pad = jnp.pad(x, ((0, N_PAD - N), (0, 0)))
    pad_i = jnp.arange(E_PAD - E, dtype=jnp.int32)
    src_pad = jnp.concatenate(
        [edge_index[0], pad_i % CH]
    ).reshape(NW, NSB, SBC, CH)
    dst_flat = jnp.concatenate(
        [edge_index[1], TRASH + pad_i % (N_ACC - N)])
    dst_seg = dst_flat.reshape(NW, NSB, SBC, CH)
    dst_deg = dst_flat.reshape(NW, NCH, CH)
    bt2 = jnp.pad(batch, (0, N_PAD - N), constant_values=G)[None, :]
    b1r, b2r, b3r, b4r = (b[None, :] for b in (b1, b2, b3, b4))
    lbr = lin_b[None, :]

    xw = pl.pallas_call(
        _tc_mm,
        grid=(GRID,),
        in_specs=[_row_spec(0), _full_spec((D, D))],
        out_specs=_row_spec(0),
        out_shape=jax.ShapeDtypeStruct((N_PAD, D), f32),
    )(x_pad, W1)
    zeros_c = jnp.zeros((CH, D), jnp.float32)
    ones_c = jnp.ones((CH, D), jnp.float32)
    degp = _sc_degree(dst_deg, zeros_c, ones_c)

    g0, dis = pl.pallas_call(
        _tc_scale,
        grid=(GRID,),
        in_specs=[_row_spec(0), _row_spec(0), _row_spec(GRID)],
        out_specs=[_row_spec(0), pl.BlockSpec((BN, 1), lambda i: (i, 0))],
        out_shape=[jax.ShapeDtypeStruct((N_PAD, D), f32),
                   jax.ShapeDtypeStruct((N_PAD, 1), f32)],
    )(xw, degp, degp)

    def seg_call(g):
        return _sc_segsum(g, src_pad, dst_seg, zeros_c)

    def layer_call(acc, g, b, w):
        return pl.pallas_call(
            _tc_layer,
            grid=(GRID,),
            in_specs=[
                _row_spec(0), _row_spec(GRID), _row_spec(0),
                pl.BlockSpec((BN, 1), lambda i: (i, 0)),
                _full_spec((1, D)), _full_spec((D, D)),
            ],
            out_specs=[_row_spec(0), _row_spec(0)],
            out_shape=[jax.ShapeDtypeStruct((N_PAD, D), f32),
                       jax.ShapeDtypeStruct((N_PAD, D), f32)],
        )(acc, acc, g, dis, b, w)

    acc0 = seg_call(g0)
    x1, g1 = layer_call(acc0, g0, b1r, W2)
    acc1 = seg_call(g1)
    x2, g2 = layer_call(acc1, g1, b2r, W3)
    acc2 = seg_call(g2)
    g3 = pl.pallas_call(
        _tc_layer_nox,
        grid=(GRID,),
        in_specs=[
            _row_spec(0), _row_spec(GRID), _row_spec(0),
            pl.BlockSpec((BN, 1), lambda i: (i, 0)),
            _full_spec((1, D)), _full_spec((D, D)),
        ],
        out_specs=_row_spec(0),
        out_shape=jax.ShapeDtypeStruct((N_PAD, D), f32),
    )(acc2, acc2, g2, dis, b3r, W4)
    acc3 = seg_call(g3)

    out = pl.pallas_call(
        _tc_pool,
        grid=(GRID,),
        in_specs=[
            _row_spec(0), _row_spec(GRID), _row_spec(0),
            pl.BlockSpec((BN, 1), lambda i: (i, 0)),
            _full_spec((1, D)),
            _row_spec(0), _row_spec(0),
            pl.BlockSpec((1, BN), lambda i: (0, i)),
            _full_spec((3 * D, NCLS)), _full_spec((1, NCLS)),
        ],
        out_specs=pl.BlockSpec((G, NCLS), lambda i: (0, 0)),
        out_shape=jax.ShapeDtypeStruct((G, NCLS), f32),
        scratch_shapes=[pltpu.VMEM((G, 3 * D), f32), pltpu.VMEM((G, 1), f32)],
    )(acc3, acc3, g3, dis, b4r, x1, x2, bt2, lin_W, lbr)

    return out


# speedup vs baseline: 1.0038x; 1.0038x over previous
import functools

import jax
import jax.numpy as jnp
from jax import lax
from jax.experimental import pallas as pl
from jax.experimental.pallas import tpu as pltpu
from jax.experimental.pallas import tpu_sc as plsc

N = 10000
E = 320000
D = 128
G = 64
NCLS = 10

N_PAD = 10240
TRASH = N
NC, NS, L = 2, 16, 16
NW = NC * NS
N_ACC = 10112
RPA = N_ACC // NS
NBUF = 2

CH = 128
NCH = 80
SBC = 8
NSB = NCH // SBC
EPW = CH * NCH
E_PAD = EPW * NW

BN = 1024
GRID = N_PAD // BN

_mesh = plsc.VectorSubcoreMesh(
    core_axis_name="c", subcore_axis_name="s", num_cores=NC, num_subcores=NS)


def _zero_stripe(acc_sh, zbuf, zrows, sid):
    full, rem = RPA // zrows, RPA % zrows
    for k in range(full):
        pltpu.sync_copy(zbuf, acc_sh.at[pl.ds(sid * RPA + k * zrows, zrows)])
    if rem:
        pltpu.sync_copy(zbuf.at[pl.ds(0, rem)],
                        acc_sh.at[pl.ds(sid * RPA + full * zrows, rem)])


def _writeback(acc_sh, out_hbm, cid, sid):
    pltpu.sync_copy(acc_sh.at[pl.ds(sid * RPA, RPA)],
                    out_hbm.at[pl.ds(cid * N_PAD + sid * RPA, RPA)])


@functools.partial(
    pl.kernel,
    out_type=jax.ShapeDtypeStruct((NC * N_PAD, D), jnp.float32),
    mesh=_mesh,
    scratch_types=[
        pltpu.VMEM((NCH, CH), jnp.int32),
        pltpu.VMEM((CH, D), jnp.float32),
        pltpu.VMEM_SHARED((N_ACC, D), jnp.float32),
    ] + [pltpu.SemaphoreType.DMA] * NBUF,
)
def _sc_degree(dst_hbm, zeros_hbm, ones_hbm, out_hbm, dst_v, buf_v, acc_sh,
               *ss):
    cid = lax.axis_index("c")
    sid = lax.axis_index("s")
    wid = sid * NC + cid

    pltpu.sync_copy(dst_hbm.at[wid], dst_v)
    pltpu.sync_copy(zeros_hbm, buf_v)
    _zero_stripe(acc_sh, buf_v, CH, sid)
    plsc.subcore_barrier()
    pltpu.sync_copy(ones_hbm, buf_v)

    for b in range(NBUF):
        pltpu.async_copy(buf_v, acc_sh.at[dst_v.at[b]], ss[b], add=True)

    @pl.loop(NBUF, NCH, step=NBUF)
    def _(j0):
        for b in range(NBUF):
            j = j0 + b
            pltpu.make_async_copy(buf_v, acc_sh.at[dst_v.at[j - NBUF]],
                                  ss[b]).wait()
            pltpu.async_copy(buf_v, acc_sh.at[dst_v.at[j]], ss[b], add=True)

    for b in range(NBUF):
        pltpu.make_async_copy(buf_v, acc_sh.at[dst_v.at[NCH - NBUF + b]],
                              ss[b]).wait()
    plsc.subcore_barrier()
    _writeback(acc_sh, out_hbm, cid, sid)


@functools.partial(
    pl.kernel,
    out_type=jax.ShapeDtypeStruct((NC * N_PAD, D), jnp.float32),
    mesh=_mesh,
    scratch_types=[
        pltpu.VMEM((2, SBC, CH), jnp.int32),
        pltpu.VMEM((2, SBC, CH), jnp.int32),
        pltpu.VMEM((CH, D), jnp.float32),
        pltpu.VMEM((CH, D), jnp.float32),
        pltpu.VMEM_SHARED((N_ACC, D), jnp.float32),
    ] + [pltpu.SemaphoreType.DMA] * 5,
)
def _sc_segsum(g_hbm, src_hbm, dst_hbm, zeros_hbm, out_hbm, src_v, dst_v,
               buf0, buf1, acc_sh, sg0, sg1, ss0, ss1, si):
    cid = lax.axis_index("c")
    sid = lax.axis_index("s")
    wid = sid * NC + cid
    bufs = (buf0, buf1)
    sg = (sg0, sg1)
    ss = (ss0, ss1)

    pltpu.sync_copy(src_hbm.at[wid, 0], src_v.at[0])
    pltpu.sync_copy(dst_hbm.at[wid, 0], dst_v.at[0])
    pltpu.async_copy(src_hbm.at[wid, 1], src_v.at[1], si)
    pltpu.async_copy(dst_hbm.at[wid, 1], dst_v.at[1], si)
    pltpu.async_copy(g_hbm.at[src_v.at[0, 0]], buf0, sg0)
    pltpu.sync_copy(zeros_hbm, buf1)
    _zero_stripe(acc_sh, buf1, CH, sid)
    plsc.subcore_barrier()
    pltpu.async_copy(g_hbm.at[src_v.at[0, 1]], buf1, sg1)
    for b in range(NBUF):
        pltpu.make_async_copy(g_hbm.at[src_v.at[0, b]], bufs[b], sg[b]).wait()
        pltpu.async_copy(bufs[b], acc_sh.at[dst_v.at[0, b]], ss[b], add=True)

    @pl.loop(NBUF, NCH, step=NBUF)
    def _(j0):
        sb = j0 // SBC
        jj = j0 % SBC
        p = sb % 2

        @pl.when(jj == 0)
        def _():
            pltpu.make_async_copy(src_hbm.at[wid, 0], src_v.at[0], si).wait()
            pltpu.make_async_copy(dst_hbm.at[wid, 0], dst_v.at[0], si).wait()

        @pl.when((jj == 0) & (sb < NSB - 1))
        def _():
            pltpu.async_copy(src_hbm.at[wid, sb + 1], src_v.at[1 - p], si)
            pltpu.async_copy(dst_hbm.at[wid, sb + 1], dst_v.at[1 - p], si)

        for b in range(NBUF):
            pltpu.make_async_copy(bufs[b], acc_sh.at[dst_v.at[0, 0]],
                                  ss[b]).wait()
            pltpu.async_copy(g_hbm.at[src_v.at[p, jj + b]], bufs[b], sg[b])
        for b in range(NBUF):
            pltpu.make_async_copy(g_hbm.at[src_v.at[p, jj + b]], bufs[b],
                                  sg[b]).wait()
            pltpu.async_copy(bufs[b], acc_sh.at[dst_v.at[p, jj + b]], ss[b],
                             add=True)

    for b in range(NBUF):
        pltpu.make_async_copy(bufs[b], acc_sh.at[dst_v.at[0, 0]], ss[b]).wait()
    plsc.subcore_barrier()
    _writeback(acc_sh, out_hbm, cid, sid)


def _tc_mm(x_ref, w_ref, o_ref):
    o_ref[...] = jnp.dot(x_ref[...], w_ref[...],
                         preferred_element_type=jnp.float32)


def _tc_scale(xw_ref, d0_ref, d1_ref, g_ref, dis_ref):
    deg = d0_ref[:, :1] + d1_ref[:, :1] + 1.0
    dis = lax.rsqrt(deg)
    dis_ref[...] = dis
    g_ref[...] = dis * xw_ref[...]


def _edge_sum(a0_ref, a1_ref, g_ref):
    return a0_ref[...] + a1_ref[...] + g_ref[...]


def _tc_layer(a0_ref, a1_ref, g_ref, dis_ref, b_ref, w_ref, h_ref, gn_ref):
    dis = dis_ref[...]
    h = jnp.maximum(
        dis * _edge_sum(a0_ref, a1_ref, g_ref) + b_ref[...], 0.0)
    h_ref[...] = h
    gn_ref[...] = dis * jnp.dot(h, w_ref[...],
                                preferred_element_type=jnp.float32)


def _tc_layer_nox(a0_ref, a1_ref, g_ref, dis_ref, b_ref, w_ref, gn_ref):
    dis = dis_ref[...]
    h = jnp.maximum(
        dis * _edge_sum(a0_ref, a1_ref, g_ref) + b_ref[...], 0.0)
    gn_ref[...] = dis * jnp.dot(h, w_ref[...],
                                preferred_element_type=jnp.float32)


def _tc_pool(a0_ref, a1_ref, g_ref, dis_ref, b_ref, x1_ref, x2_ref, bt_ref,
             lw_ref, lb_ref, out_ref, sums, cnts):
    i = pl.program_id(0)
    x3 = jnp.maximum(
        dis_ref[...] * _edge_sum(a0_ref, a1_ref, g_ref) + b_ref[...], 0.0)
    xs = jnp.concatenate([x1_ref[...], x2_ref[...], x3], axis=1)
    oh = (lax.broadcasted_iota(jnp.int32, (G, BN), 0)
          == bt_ref[...]).astype(jnp.float32)
    s_c = jnp.dot(oh, xs, preferred_element_type=jnp.float32)
    c_c = jnp.sum(oh, axis=1, keepdims=True)

    @pl.when(i == 0)
    def _():
        sums[...] = s_c
        cnts[...] = c_c

    @pl.when(i > 0)
    def _():
        sums[...] += s_c
        cnts[...] += c_c

    @pl.when(i == GRID - 1)
    def _():
        pooled = sums[...] / jnp.maximum(cnts[...], 1.0)
        logits = jnp.dot(pooled, lw_ref[...],
                         preferred_element_type=jnp.float32) + lb_ref[...]
        m = jnp.max(logits, axis=1, keepdims=True)
        lse = jnp.log(jnp.sum(jnp.exp(logits - m), axis=1, keepdims=True)) + m
        out_ref[...] = logits - lse


def _row_spec(off):
    return pl.BlockSpec((BN, D), lambda i, o=off: (i + o, 0))


def _full_spec(shape):
    return pl.BlockSpec(shape, lambda i: (0, 0))


def kernel(x, edge_index, batch, W1, b1, W2, b2, W3, b3, W4, b4, lin_W, lin_b):
    f32 = jnp.float32
    x_pad = jnp.pad(x, ((0, N_PAD - N), (0, 0)))
    pad_i = jnp.arange(E_PAD - E, dtype=jnp.int32)
    src_pad = jnp.concatenate(
        [edge_index[0], pad_i % CH]
    ).reshape(NW, NSB, SBC, CH)
    dst_flat = jnp.concatenate(
        [edge_index[1], TRASH + pad_i % (N_ACC - N)])
    dst_seg = dst_flat.reshape(NW, NSB, SBC, CH)
    dst_deg = dst_flat.reshape(NW, NCH, CH)
    bt2 = jnp.pad(batch, (0, N_PAD - N), constant_values=G)[None, :]
    b1r, b2r, b3r, b4r = (b[None, :] for b in (b1, b2, b3, b4))
    lbr = lin_b[None, :]

    xw = pl.pallas_call(
        _tc_mm,
        grid=(GRID,),
        in_specs=[_row_spec(0), _full_spec((D, D))],
        out_specs=_row_spec(0),
        out_shape=jax.ShapeDtypeStruct((N_PAD, D), f32),
    )(x_pad, W1)
    zeros_c = jnp.zeros((CH, D), jnp.float32)
    ones_c = jnp.ones((CH, D), jnp.float32)
    degp = _sc_degree(dst_deg, zeros_c, ones_c)

    g0, dis = pl.pallas_call(
        _tc_scale,
        grid=(GRID,),
        in_specs=[_row_spec(0), _row_spec(0), _row_spec(GRID)],
        out_specs=[_row_spec(0), pl.BlockSpec((BN, 1), lambda i: (i, 0))],
        out_shape=[jax.ShapeDtypeStruct((N_PAD, D), f32),
                   jax.ShapeDtypeStruct((N_PAD, 1), f32)],
    )(xw, degp, degp)

    def seg_call(g):
        return _sc_segsum(g, src_pad, dst_seg, zeros_c)

    def layer_call(acc, g, b, w):
        return pl.pallas_call(
            _tc_layer,
            grid=(GRID,),
            in_specs=[
                _row_spec(0), _row_spec(GRID), _row_spec(0),
                pl.BlockSpec((BN, 1), lambda i: (i, 0)),
                _full_spec((1, D)), _full_spec((D, D)),
            ],
            out_specs=[_row_spec(0), _row_spec(0)],
            out_shape=[jax.ShapeDtypeStruct((N_PAD, D), f32),
                       jax.ShapeDtypeStruct((N_PAD, D), f32)],
        )(acc, acc, g, dis, b, w)

    acc0 = seg_call(g0)
    x1, g1 = layer_call(acc0, g0, b1r, W2)
    acc1 = seg_call(g1)
    x2, g2 = layer_call(acc1, g1, b2r, W3)
    acc2 = seg_call(g2)
    g3 = pl.pallas_call(
        _tc_layer_nox,
        grid=(GRID,),
        in_specs=[
            _row_spec(0), _row_spec(GRID), _row_spec(0),
            pl.BlockSpec((BN, 1), lambda i: (i, 0)),
            _full_spec((1, D)), _full_spec((D, D)),
        ],
        out_specs=_row_spec(0),
        out_shape=jax.ShapeDtypeStruct((N_PAD, D), f32),
    )(acc2, acc2, g2, dis, b3r, W4)
    acc3 = seg_call(g3)

    out = pl.pallas_call(
        _tc_pool,
        grid=(GRID,),
        in_specs=[
            _row_spec(0), _row_spec(GRID), _row_spec(0),
            pl.BlockSpec((BN, 1), lambda i: (i, 0)),
            _full_spec((1, D)),
            _row_spec(0), _row_spec(0),
            pl.BlockSpec((1, BN), lambda i: (0, i)),
            _full_spec((3 * D, NCLS)), _full_spec((1, NCLS)),
        ],
        out_specs=pl.BlockSpec((G, NCLS), lambda i: (0, 0)),
        out_shape=jax.ShapeDtypeStruct((G, NCLS), f32),
        scratch_shapes=[pltpu.VMEM((G, 3 * D), f32), pltpu.VMEM((G, 1), f32)],
    )(acc3, acc3, g3, dis, b4r, x1, x2, bt2, lin_W, lbr)

    return out


# speedup vs baseline: 1.0296x; 1.0257x over previous
import functools

import jax
import jax.numpy as jnp
from jax import lax
from jax.experimental import pallas as pl
from jax.experimental.pallas import tpu as pltpu
from jax.experimental.pallas import tpu_sc as plsc

N = 10000
E = 320000
D = 128
G = 64
NCLS = 10

N_PAD = 10240
TRASH = N
NC, NS, L = 2, 16, 16
NW = NC * NS
N_ACC = 10112
RPA = N_ACC // NS
NBUF = 2

CH = 128
NCH = 80
SBC = 8
NSB = NCH // SBC
EPW = CH * NCH
E_PAD = EPW * NW

BN = 1024
GRID = N_PAD // BN

_mesh = plsc.VectorSubcoreMesh(
    core_axis_name="c", subcore_axis_name="s", num_cores=NC, num_subcores=NS)


def _fill_rows(buf, nrows, val):
    def body(i, _):
        for j in range(D // L):
            buf[i, pl.ds(j * L, L)] = jnp.full((L,), val, jnp.float32)
        return _
    lax.fori_loop(0, nrows, body, 0)


def _zero_stripe(acc_sh, zbuf, zrows, sid):
    full, rem = RPA // zrows, RPA % zrows
    for k in range(full):
        pltpu.sync_copy(zbuf, acc_sh.at[pl.ds(sid * RPA + k * zrows, zrows)])
    if rem:
        pltpu.sync_copy(zbuf.at[pl.ds(0, rem)],
                        acc_sh.at[pl.ds(sid * RPA + full * zrows, rem)])


def _writeback(acc_sh, out_hbm, cid, sid):
    pltpu.sync_copy(acc_sh.at[pl.ds(sid * RPA, RPA)],
                    out_hbm.at[pl.ds(cid * N_PAD + sid * RPA, RPA)])


@functools.partial(
    pl.kernel,
    out_type=jax.ShapeDtypeStruct((NC * N_PAD, D), jnp.float32),
    mesh=_mesh,
    scratch_types=[
        pltpu.VMEM((NCH, CH), jnp.int32),
        pltpu.VMEM((CH, D), jnp.float32),
        pltpu.VMEM_SHARED((N_ACC, D), jnp.float32),
    ] + [pltpu.SemaphoreType.DMA] * NBUF,
)
def _sc_degree(dst_hbm, out_hbm, dst_v, buf_v, acc_sh, *ss):
    cid = lax.axis_index("c")
    sid = lax.axis_index("s")
    wid = sid * NC + cid

    pltpu.sync_copy(dst_hbm.at[wid], dst_v)
    _fill_rows(buf_v, CH, 0.0)
    _zero_stripe(acc_sh, buf_v, CH, sid)
    plsc.subcore_barrier()
    _fill_rows(buf_v, CH, 1.0)

    for b in range(NBUF):
        pltpu.async_copy(buf_v, acc_sh.at[dst_v.at[b]], ss[b], add=True)

    @pl.loop(NBUF, NCH, step=NBUF)
    def _(j0):
        for b in range(NBUF):
            j = j0 + b
            pltpu.make_async_copy(buf_v, acc_sh.at[dst_v.at[j - NBUF]],
                                  ss[b]).wait()
            pltpu.async_copy(buf_v, acc_sh.at[dst_v.at[j]], ss[b], add=True)

    for b in range(NBUF):
        pltpu.make_async_copy(buf_v, acc_sh.at[dst_v.at[NCH - NBUF + b]],
                              ss[b]).wait()
    plsc.subcore_barrier()
    _writeback(acc_sh, out_hbm, cid, sid)


@functools.partial(
    pl.kernel,
    out_type=jax.ShapeDtypeStruct((NC * N_PAD, D), jnp.float32),
    mesh=_mesh,
    scratch_types=[
        pltpu.VMEM((2, SBC, CH), jnp.int32),
        pltpu.VMEM((2, SBC, CH), jnp.int32),
        pltpu.VMEM((CH, D), jnp.float32),
        pltpu.VMEM((CH, D), jnp.float32),
        pltpu.VMEM_SHARED((N_ACC, D), jnp.float32),
    ] + [pltpu.SemaphoreType.DMA] * 5,
)
def _sc_segsum(g_hbm, src_hbm, dst_hbm, out_hbm, src_v, dst_v,
               buf0, buf1, acc_sh, sg0, sg1, ss0, ss1, si):
    cid = lax.axis_index("c")
    sid = lax.axis_index("s")
    wid = sid * NC + cid
    bufs = (buf0, buf1)
    sg = (sg0, sg1)
    ss = (ss0, ss1)

    pltpu.sync_copy(src_hbm.at[wid, 0], src_v.at[0])
    pltpu.sync_copy(dst_hbm.at[wid, 0], dst_v.at[0])
    pltpu.async_copy(src_hbm.at[wid, 1], src_v.at[1], si)
    pltpu.async_copy(dst_hbm.at[wid, 1], dst_v.at[1], si)
    pltpu.async_copy(g_hbm.at[src_v.at[0, 0]], buf0, sg0)
    _fill_rows(buf1, CH, 0.0)
    _zero_stripe(acc_sh, buf1, CH, sid)
    plsc.subcore_barrier()
    pltpu.async_copy(g_hbm.at[src_v.at[0, 1]], buf1, sg1)
    for b in range(NBUF):
        pltpu.make_async_copy(g_hbm.at[src_v.at[0, b]], bufs[b], sg[b]).wait()
        pltpu.async_copy(bufs[b], acc_sh.at[dst_v.at[0, b]], ss[b], add=True)

    @pl.loop(NBUF, NCH, step=NBUF)
    def _(j0):
        sb = j0 // SBC
        jj = j0 % SBC
        p = sb % 2

        @pl.when(jj == 0)
        def _():
            pltpu.make_async_copy(src_hbm.at[wid, 0], src_v.at[0], si).wait()
            pltpu.make_async_copy(dst_hbm.at[wid, 0], dst_v.at[0], si).wait()

        @pl.when((jj == 0) & (sb < NSB - 1))
        def _():
            pltpu.async_copy(src_hbm.at[wid, sb + 1], src_v.at[1 - p], si)
            pltpu.async_copy(dst_hbm.at[wid, sb + 1], dst_v.at[1 - p], si)

        for b in range(NBUF):
            pltpu.make_async_copy(bufs[b], acc_sh.at[dst_v.at[0, 0]],
                                  ss[b]).wait()
            pltpu.async_copy(g_hbm.at[src_v.at[p, jj + b]], bufs[b], sg[b])
        for b in range(NBUF):
            pltpu.make_async_copy(g_hbm.at[src_v.at[p, jj + b]], bufs[b],
                                  sg[b]).wait()
            pltpu.async_copy(bufs[b], acc_sh.at[dst_v.at[p, jj + b]], ss[b],
                             add=True)

    for b in range(NBUF):
        pltpu.make_async_copy(bufs[b], acc_sh.at[dst_v.at[0, 0]], ss[b]).wait()
    plsc.subcore_barrier()
    _writeback(acc_sh, out_hbm, cid, sid)


def _tc_mm(x_ref, w_ref, o_ref):
    o_ref[...] = jnp.dot(x_ref[...], w_ref[...],
                         preferred_element_type=jnp.float32)


def _tc_scale(xw_ref, d0_ref, d1_ref, g_ref, dis_ref):
    deg = d0_ref[:, :1] + d1_ref[:, :1] + 1.0
    dis = lax.rsqrt(deg)
    dis_ref[...] = dis
    g_ref[...] = dis * xw_ref[...]


def _edge_sum(a0_ref, a1_ref, g_ref):
    return a0_ref[...] + a1_ref[...] + g_ref[...]


def _tc_layer(a0_ref, a1_ref, g_ref, dis_ref, b_ref, w_ref, h_ref, gn_ref):
    dis = dis_ref[...]
    h = jnp.maximum(
        dis * _edge_sum(a0_ref, a1_ref, g_ref) + b_ref[...], 0.0)
    h_ref[...] = h
    gn_ref[...] = dis * jnp.dot(h, w_ref[...],
                                preferred_element_type=jnp.float32)


def _tc_layer_nox(a0_ref, a1_ref, g_ref, dis_ref, b_ref, w_ref, gn_ref):
    dis = dis_ref[...]
    h = jnp.maximum(
        dis * _edge_sum(a0_ref, a1_ref, g_ref) + b_ref[...], 0.0)
    gn_ref[...] = dis * jnp.dot(h, w_ref[...],
                                preferred_element_type=jnp.float32)


def _tc_pool(a0_ref, a1_ref, g_ref, dis_ref, b_ref, x1_ref, x2_ref, bt_ref,
             lw_ref, lb_ref, out_ref, sums, cnts):
    i = pl.program_id(0)
    x3 = jnp.maximum(
        dis_ref[...] * _edge_sum(a0_ref, a1_ref, g_ref) + b_ref[...], 0.0)
    xs = jnp.concatenate([x1_ref[...], x2_ref[...], x3], axis=1)
    oh = (lax.broadcasted_iota(jnp.int32, (G, BN), 0)
          == bt_ref[...]).astype(jnp.float32)
    s_c = jnp.dot(oh, xs, preferred_element_type=jnp.float32)
    c_c = jnp.sum(oh, axis=1, keepdims=True)

    @pl.when(i == 0)
    def _():
        sums[...] = s_c
        cnts[...] = c_c

    @pl.when(i > 0)
    def _():
        sums[...] += s_c
        cnts[...] += c_c

    @pl.when(i == GRID - 1)
    def _():
        pooled = sums[...] / jnp.maximum(cnts[...], 1.0)
        logits = jnp.dot(pooled, lw_ref[...],
                         preferred_element_type=jnp.float32) + lb_ref[...]
        m = jnp.max(logits, axis=1, keepdims=True)
        lse = jnp.log(jnp.sum(jnp.exp(logits - m), axis=1, keepdims=True)) + m
        out_ref[...] = logits - lse


def _row_spec(off):
    return pl.BlockSpec((BN, D), lambda i, o=off: (i + o, 0))


def _full_spec(shape):
    return pl.BlockSpec(shape, lambda i: (0, 0))


def kernel(x, edge_index, batch, W1, b1, W2, b2, W3, b3, W4, b4, lin_W, lin_b):
    f32 = jnp.float32
    x_pad = jnp.pad(x, ((0, N_PAD - N), (0, 0)))
    pad_i = jnp.arange(E_PAD - E, dtype=jnp.int32)
    src_pad = jnp.concatenate(
        [edge_index[0], pad_i % CH]
    ).reshape(NW, NSB, SBC, CH)
    dst_flat = jnp.concatenate(
        [edge_index[1], TRASH + pad_i % (N_ACC - N)])
    dst_seg = dst_flat.reshape(NW, NSB, SBC, CH)
    dst_deg = dst_flat.reshape(NW, NCH, CH)
    bt2 = jnp.pad(batch, (0, N_PAD - N), constant_values=G)[None, :]
    b1r, b2r, b3r, b4r = (b[None, :] for b in (b1, b2, b3, b4))
    lbr = lin_b[None, :]

    xw = pl.pallas_call(
        _tc_mm,
        grid=(GRID,),
        in_specs=[_row_spec(0), _full_spec((D, D))],
        out_specs=_row_spec(0),
        out_shape=jax.ShapeDtypeStruct((N_PAD, D), f32),
    )(x_pad, W1)
    degp = _sc_degree(dst_deg)

    g0, dis = pl.pallas_call(
        _tc_scale,
        grid=(GRID,),
        in_specs=[_row_spec(0), _row_spec(0), _row_spec(GRID)],
        out_specs=[_row_spec(0), pl.BlockSpec((BN, 1), lambda i: (i, 0))],
        out_shape=[jax.ShapeDtypeStruct((N_PAD, D), f32),
                   jax.ShapeDtypeStruct((N_PAD, 1), f32)],
    )(xw, degp, degp)

    def seg_call(g):
        return _sc_segsum(g, src_pad, dst_seg)

    def layer_call(acc, g, b, w):
        return pl.pallas_call(
            _tc_layer,
            grid=(GRID,),
            in_specs=[
                _row_spec(0), _row_spec(GRID), _row_spec(0),
                pl.BlockSpec((BN, 1), lambda i: (i, 0)),
                _full_spec((1, D)), _full_spec((D, D)),
            ],
            out_specs=[_row_spec(0), _row_spec(0)],
            out_shape=[jax.ShapeDtypeStruct((N_PAD, D), f32),
                       jax.ShapeDtypeStruct((N_PAD, D), f32)],
        )(acc, acc, g, dis, b, w)

    acc0 = seg_call(g0)
    x1, g1 = layer_call(acc0, g0, b1r, W2)
    acc1 = seg_call(g1)
    x2, g2 = layer_call(acc1, g1, b2r, W3)
    acc2 = seg_call(g2)
    g3 = pl.pallas_call(
        _tc_layer_nox,
        grid=(GRID,),
        in_specs=[
            _row_spec(0), _row_spec(GRID), _row_spec(0),
            pl.BlockSpec((BN, 1), lambda i: (i, 0)),
            _full_spec((1, D)), _full_spec((D, D)),
        ],
        out_specs=_row_spec(0),
        out_shape=jax.ShapeDtypeStruct((N_PAD, D), f32),
    )(acc2, acc2, g2, dis, b3r, W4)
    acc3 = seg_call(g3)

    out = pl.pallas_call(
        _tc_pool,
        grid=(GRID,),
        in_specs=[
            _row_spec(0), _row_spec(GRID), _row_spec(0),
            pl.BlockSpec((BN, 1), lambda i: (i, 0)),
            _full_spec((1, D)),
            _row_spec(0), _row_spec(0),
            pl.BlockSpec((1, BN), lambda i: (0, i)),
            _full_spec((3 * D, NCLS)), _full_spec((1, NCLS)),
        ],
        out_specs=pl.BlockSpec((G, NCLS), lambda i: (0, 0)),
        out_shape=jax.ShapeDtypeStruct((G, NCLS), f32),
        scratch_shapes=[pltpu.VMEM((G, 3 * D), f32), pltpu.VMEM((G, 1), f32)],
    )(acc3, acc3, g3, dis, b4r, x1, x2, bt2, lin_W, lbr)

    return out


# speedup vs baseline: 1.0312x; 1.0015x over previous
import functools

import jax
import jax.numpy as jnp
from jax import lax
from jax.experimental import pallas as pl
from jax.experimental.pallas import tpu as pltpu
from jax.experimental.pallas import tpu_sc as plsc

N = 10000
E = 320000
D = 128
G = 64
NCLS = 10

N_PAD = 10240
TRASH = N
NC, NS, L = 2, 16, 16
NW = NC * NS
N_ACC = 10112
RPA = N_ACC // NS
NBUF = 2

CH = 128
NCH = 80
SBC = 8
NSB = NCH // SBC
EPW = CH * NCH
E_PAD = EPW * NW

BN = 1024
GRID = N_PAD // BN

_mesh = plsc.VectorSubcoreMesh(
    core_axis_name="c", subcore_axis_name="s", num_cores=NC, num_subcores=NS)


def _fill_rows(buf, nrows, val):
    def body(i, _):
        for j in range(D // L):
            buf[i, pl.ds(j * L, L)] = jnp.full((L,), val, jnp.float32)
        return _
    lax.fori_loop(0, nrows, body, 0)


def _zero_stripe(acc_sh, zbuf, zrows, sid):
    full, rem = RPA // zrows, RPA % zrows
    for k in range(full):
        pltpu.sync_copy(zbuf, acc_sh.at[pl.ds(sid * RPA + k * zrows, zrows)])
    if rem:
        pltpu.sync_copy(zbuf.at[pl.ds(0, rem)],
                        acc_sh.at[pl.ds(sid * RPA + full * zrows, rem)])


def _writeback(acc_sh, out_hbm, cid, sid):
    pltpu.sync_copy(acc_sh.at[pl.ds(sid * RPA, RPA)],
                    out_hbm.at[pl.ds(cid * N_PAD + sid * RPA, RPA)])


@functools.partial(
    pl.kernel,
    out_type=jax.ShapeDtypeStruct((NC * N_PAD, D), jnp.float32),
    mesh=_mesh,
    scratch_types=[
        pltpu.VMEM((NCH, CH), jnp.int32),
        pltpu.VMEM((CH, D), jnp.float32),
        pltpu.VMEM_SHARED((N_ACC, D), jnp.float32),
    ] + [pltpu.SemaphoreType.DMA] * NBUF,
)
def _sc_degree(dst_hbm, out_hbm, dst_v, buf_v, acc_sh, *ss):
    cid = lax.axis_index("c")
    sid = lax.axis_index("s")
    wid = sid * NC + cid

    pltpu.sync_copy(dst_hbm.at[wid], dst_v)
    _fill_rows(buf_v, CH, 0.0)
    _zero_stripe(acc_sh, buf_v, CH, sid)
    plsc.subcore_barrier()
    _fill_rows(buf_v, CH, 1.0)

    for b in range(NBUF):
        pltpu.async_copy(buf_v, acc_sh.at[dst_v.at[b]], ss[b], add=True)

    @pl.loop(NBUF, NCH, step=NBUF)
    def _(j0):
        for b in range(NBUF):
            j = j0 + b
            pltpu.make_async_copy(buf_v, acc_sh.at[dst_v.at[j - NBUF]],
                                  ss[b]).wait()
            pltpu.async_copy(buf_v, acc_sh.at[dst_v.at[j]], ss[b], add=True)

    for b in range(NBUF):
        pltpu.make_async_copy(buf_v, acc_sh.at[dst_v.at[NCH - NBUF + b]],
                              ss[b]).wait()
    plsc.subcore_barrier()
    _writeback(acc_sh, out_hbm, cid, sid)


@functools.partial(
    pl.kernel,
    out_type=jax.ShapeDtypeStruct((NC * N_PAD, D), jnp.float32),
    mesh=_mesh,
    scratch_types=[
        pltpu.VMEM((2, SBC, CH), jnp.int32),
        pltpu.VMEM((2, SBC, CH), jnp.int32),
        pltpu.VMEM((CH, D), jnp.float32),
        pltpu.VMEM((CH, D), jnp.float32),
        pltpu.VMEM_SHARED((N_ACC, D), jnp.float32),
    ] + [pltpu.SemaphoreType.DMA] * 5,
)
def _sc_segsum(g_hbm, src_hbm, dst_hbm, out_hbm, src_v, dst_v,
               buf0, buf1, acc_sh, sg0, sg1, ss0, ss1, si):
    cid = lax.axis_index("c")
    sid = lax.axis_index("s")
    wid = sid * NC + cid
    bufs = (buf0, buf1)
    sg = (sg0, sg1)
    ss = (ss0, ss1)

    pltpu.sync_copy(src_hbm.at[wid, 0], src_v.at[0])
    pltpu.sync_copy(dst_hbm.at[wid, 0], dst_v.at[0])
    pltpu.async_copy(src_hbm.at[wid, 1], src_v.at[1], si)
    pltpu.async_copy(dst_hbm.at[wid, 1], dst_v.at[1], si)
    pltpu.async_copy(g_hbm.at[src_v.at[0, 0]], buf0, sg0)
    _fill_rows(buf1, CH, 0.0)
    _zero_stripe(acc_sh, buf1, CH, sid)
    plsc.subcore_barrier()
    pltpu.async_copy(g_hbm.at[src_v.at[0, 1]], buf1, sg1)
    for b in range(NBUF):
        pltpu.make_async_copy(g_hbm.at[src_v.at[0, b]], bufs[b], sg[b]).wait()
        pltpu.async_copy(bufs[b], acc_sh.at[dst_v.at[0, b]], ss[b], add=True)

    @pl.loop(NBUF, NCH, step=NBUF)
    def _(j0):
        sb = j0 // SBC
        jj = j0 % SBC
        p = sb % 2

        @pl.when(jj == 0)
        def _():
            pltpu.make_async_copy(src_hbm.at[wid, 0], src_v.at[0], si).wait()
            pltpu.make_async_copy(dst_hbm.at[wid, 0], dst_v.at[0], si).wait()

        @pl.when((jj == 0) & (sb < NSB - 1))
        def _():
            pltpu.async_copy(src_hbm.at[wid, sb + 1], src_v.at[1 - p], si)
            pltpu.async_copy(dst_hbm.at[wid, sb + 1], dst_v.at[1 - p], si)

        for b in range(NBUF):
            pltpu.make_async_copy(bufs[b], acc_sh.at[dst_v.at[0, 0]],
                                  ss[b]).wait()
            pltpu.async_copy(g_hbm.at[src_v.at[p, jj + b, pl.ds(0, CH // 2)]],
                             bufs[b].at[pl.ds(0, CH // 2)], sg[b])
            pltpu.async_copy(
                g_hbm.at[src_v.at[p, jj + b, pl.ds(CH // 2, CH // 2)]],
                bufs[b].at[pl.ds(CH // 2, CH // 2)], sg[b])
        for b in range(NBUF):
            pltpu.make_async_copy(g_hbm.at[src_v.at[p, jj + b,
                                                    pl.ds(0, CH // 2)]],
                                  bufs[b].at[pl.ds(0, CH // 2)], sg[b]).wait()
            pltpu.make_async_copy(g_hbm.at[src_v.at[p, jj + b,
                                                    pl.ds(CH // 2, CH // 2)]],
                                  bufs[b].at[pl.ds(CH // 2, CH // 2)],
                                  sg[b]).wait()
            pltpu.async_copy(bufs[b], acc_sh.at[dst_v.at[p, jj + b]], ss[b],
                             add=True)

    for b in range(NBUF):
        pltpu.make_async_copy(bufs[b], acc_sh.at[dst_v.at[0, 0]], ss[b]).wait()
    plsc.subcore_barrier()
    _writeback(acc_sh, out_hbm, cid, sid)


def _tc_mm(x_ref, w_ref, o_ref):
    o_ref[...] = jnp.dot(x_ref[...], w_ref[...],
                         preferred_element_type=jnp.float32)


def _tc_scale(xw_ref, d0_ref, d1_ref, g_ref, dis_ref):
    deg = d0_ref[:, :1] + d1_ref[:, :1] + 1.0
    dis = lax.rsqrt(deg)
    dis_ref[...] = dis
    g_ref[...] = dis * xw_ref[...]


def _edge_sum(a0_ref, a1_ref, g_ref):
    return a0_ref[...] + a1_ref[...] + g_ref[...]


def _tc_layer(a0_ref, a1_ref, g_ref, dis_ref, b_ref, w_ref, h_ref, gn_ref):
    dis = dis_ref[...]
    h = jnp.maximum(
        dis * _edge_sum(a0_ref, a1_ref, g_ref) + b_ref[...], 0.0)
    h_ref[...] = h
    gn_ref[...] = dis * jnp.dot(h, w_ref[...],
                                preferred_element_type=jnp.float32)


def _tc_layer_nox(a0_ref, a1_ref, g_ref, dis_ref, b_ref, w_ref, gn_ref):
    dis = dis_ref[...]
    h = jnp.maximum(
        dis * _edge_sum(a0_ref, a1_ref, g_ref) + b_ref[...], 0.0)
    gn_ref[...] = dis * jnp.dot(h, w_ref[...],
                                preferred_element_type=jnp.float32)


def _tc_pool(a0_ref, a1_ref, g_ref, dis_ref, b_ref, x1_ref, x2_ref, bt_ref,
             lw_ref, lb_ref, out_ref, sums, cnts):
    i = pl.program_id(0)
    x3 = jnp.maximum(
        dis_ref[...] * _edge_sum(a0_ref, a1_ref, g_ref) + b_ref[...], 0.0)
    xs = jnp.concatenate([x1_ref[...], x2_ref[...], x3], axis=1)
    oh = (lax.broadcasted_iota(jnp.int32, (G, BN), 0)
          == bt_ref[...]).astype(jnp.float32)
    s_c = jnp.dot(oh, xs, preferred_element_type=jnp.float32)
    c_c = jnp.sum(oh, axis=1, keepdims=True)

    @pl.when(i == 0)
    def _():
        sums[...] = s_c
        cnts[...] = c_c

    @pl.when(i > 0)
    def _():
        sums[...] += s_c
        cnts[...] += c_c

    @pl.when(i == GRID - 1)
    def _():
        pooled = sums[...] / jnp.maximum(cnts[...], 1.0)
        logits = jnp.dot(pooled, lw_ref[...],
                         preferred_element_type=jnp.float32) + lb_ref[...]
        m = jnp.max(logits, axis=1, keepdims=True)
        lse = jnp.log(jnp.sum(jnp.exp(logits - m), axis=1, keepdims=True)) + m
        out_ref[...] = logits - lse


def _row_spec(off):
    return pl.BlockSpec((BN, D), lambda i, o=off: (i + o, 0))


def _full_spec(shape):
    return pl.BlockSpec(shape, lambda i: (0, 0))


def kernel(x, edge_index, batch, W1, b1, W2, b2, W3, b3, W4, b4, lin_W, lin_b):
    f32 = jnp.float32
    x_pad = jnp.pad(x, ((0, N_PAD - N), (0, 0)))
    pad_i = jnp.arange(E_PAD - E, dtype=jnp.int32)
    src_pad = jnp.concatenate(
        [edge_index[0], pad_i % CH]
    ).reshape(NW, NSB, SBC, CH)
    dst_flat = jnp.concatenate(
        [edge_index[1], TRASH + pad_i % (N_ACC - N)])
    dst_seg = dst_flat.reshape(NW, NSB, SBC, CH)
    dst_deg = dst_flat.reshape(NW, NCH, CH)
    bt2 = jnp.pad(batch, (0, N_PAD - N), constant_values=G)[None, :]
    b1r, b2r, b3r, b4r = (b[None, :] for b in (b1, b2, b3, b4))
    lbr = lin_b[None, :]

    xw = pl.pallas_call(
        _tc_mm,
        grid=(GRID,),
        in_specs=[_row_spec(0), _full_spec((D, D))],
        out_specs=_row_spec(0),
        out_shape=jax.ShapeDtypeStruct((N_PAD, D), f32),
    )(x_pad, W1)
    degp = _sc_degree(dst_deg)

    g0, dis = pl.pallas_call(
        _tc_scale,
        grid=(GRID,),
        in_specs=[_row_spec(0), _row_spec(0), _row_spec(GRID)],
        out_specs=[_row_spec(0), pl.BlockSpec((BN, 1), lambda i: (i, 0))],
        out_shape=[jax.ShapeDtypeStruct((N_PAD, D), f32),
                   jax.ShapeDtypeStruct((N_PAD, 1), f32)],
    )(xw, degp, degp)

    def seg_call(g):
        return _sc_segsum(g, src_pad, dst_seg)

    def layer_call(acc, g, b, w):
        return pl.pallas_call(
            _tc_layer,
            grid=(GRID,),
            in_specs=[
                _row_spec(0), _row_spec(GRID), _row_spec(0),
                pl.BlockSpec((BN, 1), lambda i: (i, 0)),
                _full_spec((1, D)), _full_spec((D, D)),
            ],
            out_specs=[_row_spec(0), _row_spec(0)],
            out_shape=[jax.ShapeDtypeStruct((N_PAD, D), f32),
                       jax.ShapeDtypeStruct((N_PAD, D), f32)],
        )(acc, acc, g, dis, b, w)

    acc0 = seg_call(g0)
    x1, g1 = layer_call(acc0, g0, b1r, W2)
    acc1 = seg_call(g1)
    x2, g2 = layer_call(acc1, g1, b2r, W3)
    acc2 = seg_call(g2)
    g3 = pl.pallas_call(
        _tc_layer_nox,
        grid=(GRID,),
        in_specs=[
            _row_spec(0), _row_spec(GRID), _row_spec(0),
            pl.BlockSpec((BN, 1), lambda i: (i, 0)),
            _full_spec((1, D)), _full_spec((D, D)),
        ],
        out_specs=_row_spec(0),
        out_shape=jax.ShapeDtypeStruct((N_PAD, D), f32),
    )(acc2, acc2, g2, dis, b3r, W4)
    acc3 = seg_call(g3)

    out = pl.pallas_call(
        _tc_pool,
        grid=(GRID,),
        in_specs=[
            _row_spec(0), _row_spec(GRID), _row_spec(0),
            pl.BlockSpec((BN, 1), lambda i: (i, 0)),
            _full_spec((1, D)),
            _row_spec(0), _row_spec(0),
            pl.BlockSpec((1, BN), lambda i: (0, i)),
            _full_spec((3 * D, NCLS)), _full_spec((1, NCLS)),
        ],
        out_specs=pl.BlockSpec((G, NCLS), lambda i: (0, 0)),
        out_shape=jax.ShapeDtypeStruct((G, NCLS), f32),
        scratch_shapes=[pltpu.VMEM((G, 3 * D), f32), pltpu.VMEM((G, 1), f32)],
    )(acc3, acc3, g3, dis, b4r, x1, x2, bt2, lin_W, lbr)

    return out


# speedup vs baseline: 1.0367x; 1.0054x over previous
import functools

import jax
import jax.numpy as jnp
from jax import lax
from jax.experimental import pallas as pl
from jax.experimental.pallas import tpu as pltpu
from jax.experimental.pallas import tpu_sc as plsc

N = 10000
E = 320000
D = 128
G = 64
NCLS = 10

N_PAD = 10240
TRASH = N
NC, NS, L = 2, 16, 16
NW = NC * NS
N_ACC = 10112
RPA = N_ACC // NS
NBUF = 2

CH = 128
NCH = 80
SBC = 8
NSB = NCH // SBC
EPW = CH * NCH
E_PAD = EPW * NW

BN = 1024
GRID = N_PAD // BN

_mesh = plsc.VectorSubcoreMesh(
    core_axis_name="c", subcore_axis_name="s", num_cores=NC, num_subcores=NS)


def _fill_rows(buf, nrows, val):
    def body(i, _):
        for j in range(D // L):
            buf[i, pl.ds(j * L, L)] = jnp.full((L,), val, jnp.float32)
        return _
    lax.fori_loop(0, nrows, body, 0)


def _zero_stripe(acc_sh, zbuf, zrows, sid):
    full, rem = RPA // zrows, RPA % zrows
    for k in range(full):
        pltpu.sync_copy(zbuf, acc_sh.at[pl.ds(sid * RPA + k * zrows, zrows)])
    if rem:
        pltpu.sync_copy(zbuf.at[pl.ds(0, rem)],
                        acc_sh.at[pl.ds(sid * RPA + full * zrows, rem)])


def _writeback(acc_sh, out_hbm, cid, sid):
    pltpu.sync_copy(acc_sh.at[pl.ds(sid * RPA, RPA)],
                    out_hbm.at[pl.ds(cid * N_PAD + sid * RPA, RPA)])


@functools.partial(
    pl.kernel,
    out_type=jax.ShapeDtypeStruct((NC * N_PAD, D), jnp.float32),
    mesh=_mesh,
    scratch_types=[
        pltpu.VMEM((NCH, CH), jnp.int32),
        pltpu.VMEM((CH, D), jnp.float32),
        pltpu.VMEM_SHARED((N_ACC, D), jnp.float32),
    ] + [pltpu.SemaphoreType.DMA] * NBUF,
)
def _sc_degree(dst_hbm, out_hbm, dst_v, buf_v, acc_sh, *ss):
    cid = lax.axis_index("c")
    sid = lax.axis_index("s")
    wid = sid * NC + cid

    pltpu.sync_copy(dst_hbm.at[wid], dst_v)
    _fill_rows(buf_v, CH, 0.0)
    _zero_stripe(acc_sh, buf_v, CH, sid)
    plsc.subcore_barrier()
    _fill_rows(buf_v, CH, 1.0)

    for b in range(NBUF):
        pltpu.async_copy(buf_v, acc_sh.at[dst_v.at[b]], ss[b], add=True)

    @pl.loop(NBUF, NCH, step=NBUF)
    def _(j0):
        for b in range(NBUF):
            j = j0 + b
            pltpu.make_async_copy(buf_v, acc_sh.at[dst_v.at[j - NBUF]],
                                  ss[b]).wait()
            pltpu.async_copy(buf_v, acc_sh.at[dst_v.at[j]], ss[b], add=True)

    for b in range(NBUF):
        pltpu.make_async_copy(buf_v, acc_sh.at[dst_v.at[NCH - NBUF + b]],
                              ss[b]).wait()
    plsc.subcore_barrier()
    _writeback(acc_sh, out_hbm, cid, sid)


@functools.partial(
    pl.kernel,
    out_type=jax.ShapeDtypeStruct((NC * N_PAD, D), jnp.float32),
    mesh=_mesh,
    scratch_types=[
        pltpu.VMEM((2, SBC, CH), jnp.int32),
        pltpu.VMEM((2, SBC, CH), jnp.int32),
        pltpu.VMEM((CH, D), jnp.float32),
        pltpu.VMEM((CH, D), jnp.float32),
        pltpu.VMEM_SHARED((N_ACC, D), jnp.float32),
    ] + [pltpu.SemaphoreType.DMA] * 5,
)
def _sc_segsum(g_hbm, src_hbm, dst_hbm, out_hbm, src_v, dst_v,
               buf0, buf1, acc_sh, sg0, sg1, ss0, ss1, si):
    cid = lax.axis_index("c")
    sid = lax.axis_index("s")
    wid = sid * NC + cid
    bufs = (buf0, buf1)
    sg = (sg0, sg1)
    ss = (ss0, ss1)

    pltpu.sync_copy(src_hbm.at[wid, 0], src_v.at[0])
    pltpu.sync_copy(dst_hbm.at[wid, 0], dst_v.at[0])
    pltpu.async_copy(src_hbm.at[wid, 1], src_v.at[1], si)
    pltpu.async_copy(dst_hbm.at[wid, 1], dst_v.at[1], si)
    pltpu.async_copy(g_hbm.at[src_v.at[0, 0]], buf0, sg0)
    _fill_rows(buf1, CH, 0.0)
    _zero_stripe(acc_sh, buf1, CH, sid)
    plsc.subcore_barrier()
    pltpu.async_copy(g_hbm.at[src_v.at[0, 1]], buf1, sg1)
    for b in range(NBUF):
        pltpu.make_async_copy(g_hbm.at[src_v.at[0, b]], bufs[b], sg[b]).wait()
        pltpu.async_copy(bufs[b], acc_sh.at[dst_v.at[0, b]], ss[b], add=True)

    @pl.loop(NBUF, NCH, step=NBUF)
    def _(j0):
        sb = j0 // SBC
        jj = j0 % SBC
        p = sb % 2

        @pl.when(jj == 0)
        def _():
            pltpu.make_async_copy(src_hbm.at[wid, 0], src_v.at[0], si).wait()
            pltpu.make_async_copy(dst_hbm.at[wid, 0], dst_v.at[0], si).wait()

        @pl.when((jj == 0) & (sb < NSB - 1))
        def _():
            pltpu.async_copy(src_hbm.at[wid, sb + 1], src_v.at[1 - p], si)
            pltpu.async_copy(dst_hbm.at[wid, sb + 1], dst_v.at[1 - p], si)

        for b in range(NBUF):
            pltpu.make_async_copy(bufs[b], acc_sh.at[dst_v.at[0, 0]],
                                  ss[b]).wait()
            pltpu.async_copy(g_hbm.at[src_v.at[p, jj + b]], bufs[b], sg[b])
        for b in range(NBUF):
            pltpu.make_async_copy(g_hbm.at[src_v.at[p, jj + b]], bufs[b],
                                  sg[b]).wait()
            pltpu.async_copy(bufs[b], acc_sh.at[dst_v.at[p, jj + b]], ss[b],
                             add=True)

    for b in range(NBUF):
        pltpu.make_async_copy(bufs[b], acc_sh.at[dst_v.at[0, 0]], ss[b]).wait()
    plsc.subcore_barrier()
    _writeback(acc_sh, out_hbm, cid, sid)


def _tc_mm(x_ref, w_ref, o_ref):
    o_ref[...] = jnp.dot(x_ref[...], w_ref[...],
                         preferred_element_type=jnp.float32)


def _tc_scale(xw_ref, d0_ref, d1_ref, g_ref, dis_ref):
    deg = d0_ref[:, :1] + d1_ref[:, :1] + 1.0
    dis = lax.rsqrt(deg)
    dis_ref[...] = dis
    g_ref[...] = dis * xw_ref[...]


def _edge_sum(a0_ref, a1_ref, g_ref):
    return a0_ref[...] + a1_ref[...] + g_ref[...]


def _tc_layer(a0_ref, a1_ref, g_ref, dis_ref, b_ref, w_ref, bt_ref,
              gn_ref, pool_ref, psum):
    i = pl.program_id(0)
    dis = dis_ref[...]
    h = jnp.maximum(
        dis * _edge_sum(a0_ref, a1_ref, g_ref) + b_ref[...], 0.0)
    gn_ref[...] = dis * jnp.dot(h, w_ref[...],
                                preferred_element_type=jnp.float32)
    oh = (lax.broadcasted_iota(jnp.int32, (G, BN), 0)
          == bt_ref[...]).astype(jnp.float32)
    p_c = jnp.dot(oh, h, preferred_element_type=jnp.float32)

    @pl.when(i == 0)
    def _():
        psum[...] = p_c

    @pl.when(i > 0)
    def _():
        psum[...] += p_c

    @pl.when(i == GRID - 1)
    def _():
        pool_ref[...] = psum[...]


def _tc_layer_nox(a0_ref, a1_ref, g_ref, dis_ref, b_ref, w_ref, gn_ref):
    dis = dis_ref[...]
    h = jnp.maximum(
        dis * _edge_sum(a0_ref, a1_ref, g_ref) + b_ref[...], 0.0)
    gn_ref[...] = dis * jnp.dot(h, w_ref[...],
                                preferred_element_type=jnp.float32)


def _tc_pool(a0_ref, a1_ref, g_ref, dis_ref, b_ref, p1_ref, p2_ref, bt_ref,
             lw_ref, lb_ref, out_ref, sums, cnts):
    i = pl.program_id(0)
    x3 = jnp.maximum(
        dis_ref[...] * _edge_sum(a0_ref, a1_ref, g_ref) + b_ref[...], 0.0)
    oh = (lax.broadcasted_iota(jnp.int32, (G, BN), 0)
          == bt_ref[...]).astype(jnp.float32)
    s_c = jnp.dot(oh, x3, preferred_element_type=jnp.float32)
    c_c = jnp.sum(oh, axis=1, keepdims=True)

    @pl.when(i == 0)
    def _():
        sums[...] = s_c
        cnts[...] = c_c

    @pl.when(i > 0)
    def _():
        sums[...] += s_c
        cnts[...] += c_c

    @pl.when(i == GRID - 1)
    def _():
        pooled = (jnp.concatenate([p1_ref[...], p2_ref[...], sums[...]],
                                  axis=1)
                  / jnp.maximum(cnts[...], 1.0))
        logits = jnp.dot(pooled, lw_ref[...],
                         preferred_element_type=jnp.float32) + lb_ref[...]
        m = jnp.max(logits, axis=1, keepdims=True)
        lse = jnp.log(jnp.sum(jnp.exp(logits - m), axis=1, keepdims=True)) + m
        out_ref[...] = logits - lse


def _row_spec(off):
    return pl.BlockSpec((BN, D), lambda i, o=off: (i + o, 0))


def _full_spec(shape):
    return pl.BlockSpec(shape, lambda i: (0, 0))


def kernel(x, edge_index, batch, W1, b1, W2, b2, W3, b3, W4, b4, lin_W, lin_b):
    f32 = jnp.float32
    x_pad = jnp.pad(x, ((0, N_PAD - N), (0, 0)))
    pad_i = jnp.arange(E_PAD - E, dtype=jnp.int32)
    src_pad = jnp.concatenate(
        [edge_index[0], pad_i % CH]
    ).reshape(NW, NSB, SBC, CH)
    dst_flat = jnp.concatenate(
        [edge_index[1], TRASH + pad_i % (N_ACC - N)])
    dst_seg = dst_flat.reshape(NW, NSB, SBC, CH)
    dst_deg = dst_flat.reshape(NW, NCH, CH)
    bt2 = jnp.pad(batch, (0, N_PAD - N), constant_values=G)[None, :]
    b1r, b2r, b3r, b4r = (b[None, :] for b in (b1, b2, b3, b4))
    lbr = lin_b[None, :]

    xw = pl.pallas_call(
        _tc_mm,
        grid=(GRID,),
        in_specs=[_row_spec(0), _full_spec((D, D))],
        out_specs=_row_spec(0),
        out_shape=jax.ShapeDtypeStruct((N_PAD, D), f32),
    )(x_pad, W1)
    degp = _sc_degree(dst_deg)

    g0, dis = pl.pallas_call(
        _tc_scale,
        grid=(GRID,),
        in_specs=[_row_spec(0), _row_spec(0), _row_spec(GRID)],
        out_specs=[_row_spec(0), pl.BlockSpec((BN, 1), lambda i: (i, 0))],
        out_shape=[jax.ShapeDtypeStruct((N_PAD, D), f32),
                   jax.ShapeDtypeStruct((N_PAD, 1), f32)],
    )(xw, degp, degp)

    def seg_call(g):
        return _sc_segsum(g, src_pad, dst_seg)

    def layer_call(acc, g, b, w):
        return pl.pallas_call(
            _tc_layer,
            grid=(GRID,),
            in_specs=[
                _row_spec(0), _row_spec(GRID), _row_spec(0),
                pl.BlockSpec((BN, 1), lambda i: (i, 0)),
                _full_spec((1, D)), _full_spec((D, D)),
                pl.BlockSpec((1, BN), lambda i: (0, i)),
            ],
            out_specs=[_row_spec(0), pl.BlockSpec((G, D), lambda i: (0, 0))],
            out_shape=[jax.ShapeDtypeStruct((N_PAD, D), f32),
                       jax.ShapeDtypeStruct((G, D), f32)],
            scratch_shapes=[pltpu.VMEM((G, D), f32)],
        )(acc, acc, g, dis, b, w, bt2)

    acc0 = seg_call(g0)
    g1, pool1 = layer_call(acc0, g0, b1r, W2)
    acc1 = seg_call(g1)
    g2, pool2 = layer_call(acc1, g1, b2r, W3)
    acc2 = seg_call(g2)
    g3 = pl.pallas_call(
        _tc_layer_nox,
        grid=(GRID,),
        in_specs=[
            _row_spec(0), _row_spec(GRID), _row_spec(0),
            pl.BlockSpec((BN, 1), lambda i: (i, 0)),
            _full_spec((1, D)), _full_spec((D, D)),
        ],
        out_specs=_row_spec(0),
        out_shape=jax.ShapeDtypeStruct((N_PAD, D), f32),
    )(acc2, acc2, g2, dis, b3r, W4)
    acc3 = seg_call(g3)

    out = pl.pallas_call(
        _tc_pool,
        grid=(GRID,),
        in_specs=[
            _row_spec(0), _row_spec(GRID), _row_spec(0),
            pl.BlockSpec((BN, 1), lambda i: (i, 0)),
            _full_spec((1, D)),
            _full_spec((G, D)), _full_spec((G, D)),
            pl.BlockSpec((1, BN), lambda i: (0, i)),
            _full_spec((3 * D, NCLS)), _full_spec((1, NCLS)),
        ],
        out_specs=pl.BlockSpec((G, NCLS), lambda i: (0, 0)),
        out_shape=jax.ShapeDtypeStruct((G, NCLS), f32),
        scratch_shapes=[pltpu.VMEM((G, D), f32), pltpu.VMEM((G, 1), f32)],
    )(acc3, acc3, g3, dis, b4r, pool1, pool2, bt2, lin_W, lbr)

    return out
